# Initial kernel scaffold; baseline (speedup 1.0000x reference)
import jax
import jax.numpy as jnp
from jax import lax
from jax.experimental import pallas as pl
from jax.experimental.pallas import tpu as pltpu
from jax.experimental.pallas import tpu_sc as plsc

VOCAB = 1000000
EMBED_DIM = 64
FEATURE_LEN = 26
BATCH = 4096
HIST = 50
LAYER1 = 128
CLASS_SIZE = 100

NUM_WORKERS = 32
R_TOT = BATCH * FEATURE_LEN
R_PER_W = R_TOT // NUM_WORKERS
CHUNK = 128
NCHUNK = R_PER_W // CHUNK

_SC_MESH = plsc.VectorSubcoreMesh(core_axis_name="c", subcore_axis_name="s")


def _sc_pool_body(emb_hbm, idx_hbm, out_hbm, idx_v, acc_v, sem_g):
    wid = lax.axis_index("s") * 2 + lax.axis_index("c")

    zero = jnp.zeros((16,), jnp.float32)

    @pl.loop(0, NCHUNK)
    def _chunk(c):
        pltpu.sync_copy(idx_hbm.at[wid, c], idx_v)

        @pl.loop(0, CHUNK)
        def _zero(i):
            for j in range(EMBED_DIM // 16):
                acc_v[i, pl.ds(j * 16, 16)] = zero

        @pl.loop(0, HIST)
        def _fire(h):
            pltpu.async_copy(emb_hbm.at[idx_v.at[h]], acc_v, sem_g, add=True)

        @pl.loop(0, HIST)
        def _drain(h):
            pltpu.make_async_copy(emb_hbm.at[idx_v.at[h]], acc_v, sem_g).wait()

        row = wid * R_PER_W + c * CHUNK
        pltpu.sync_copy(acc_v, out_hbm.at[pl.ds(row, CHUNK)])


@jax.jit
def _sc_pool(emb, idx4):
    k = pl.kernel(
        _sc_pool_body,
        out_type=jax.ShapeDtypeStruct((R_TOT, EMBED_DIM), jnp.float32),
        mesh=_SC_MESH,
        scratch_types=[
            pltpu.VMEM((HIST, CHUNK), jnp.int32),
            pltpu.VMEM((CHUNK, EMBED_DIM), jnp.float32),
            pltpu.SemaphoreType.DMA,
        ],
    )
    return k(emb, idx4)


BB = 256


def _tc_mlp_body(x_ref, w1_ref, b1_ref, w2_ref, b2_ref, o_ref):
    x = x_ref[...] * (1.0 / BATCH)
    h = jnp.dot(x, w1_ref[...], preferred_element_type=jnp.float32) + b1_ref[...]
    h = jnp.maximum(h, 0.0)
    o = jnp.dot(h, w2_ref[...], preferred_element_type=jnp.float32) + b2_ref[...]
    m = jnp.max(o, axis=1, keepdims=True)
    e = jnp.exp(o - m)
    lse = jnp.log(jnp.sum(e, axis=1, keepdims=True)) + m
    o_ref[...] = o - lse


@jax.jit
def _tc_mlp(x, W1, b1, W2, b2):
    d_in = FEATURE_LEN * EMBED_DIM
    return pl.pallas_call(
        _tc_mlp_body,
        grid=(BATCH // BB,),
        in_specs=[
            pl.BlockSpec((BB, d_in), lambda i: (i, 0)),
            pl.BlockSpec((d_in, LAYER1), lambda i: (0, 0)),
            pl.BlockSpec((1, LAYER1), lambda i: (0, 0)),
            pl.BlockSpec((LAYER1, CLASS_SIZE), lambda i: (0, 0)),
            pl.BlockSpec((1, CLASS_SIZE), lambda i: (0, 0)),
        ],
        out_specs=pl.BlockSpec((BB, CLASS_SIZE), lambda i: (i, 0)),
        out_shape=jax.ShapeDtypeStruct((BATCH, CLASS_SIZE), jnp.float32),
    )(x, W1, b1, W2, b2)


def kernel(batch_inputs, emb, W1, b1, W2, b2):
    idx = batch_inputs.astype(jnp.int32)
    a = idx.transpose(1, 0, 2).reshape(R_TOT, HIST)
    idx4 = a.reshape(NUM_WORKERS, NCHUNK, CHUNK, HIST).swapaxes(2, 3)

    pooled = _sc_pool(emb, idx4)
    x = pooled.reshape(BATCH, FEATURE_LEN * EMBED_DIM)
    return _tc_mlp(x, W1, b1.reshape(1, -1), W2, b2.reshape(1, -1))


# speedup vs baseline: 8.5318x; 8.5318x over previous
import jax
import jax.numpy as jnp
from jax import lax
from jax.experimental import pallas as pl
from jax.experimental.pallas import tpu as pltpu
from jax.experimental.pallas import tpu_sc as plsc

VOCAB = 1000000
EMBED_DIM = 64
FEATURE_LEN = 26
BATCH = 4096
HIST = 50
LAYER1 = 128
CLASS_SIZE = 100

NUM_WORKERS = 32
R_TOT = BATCH * FEATURE_LEN
R_PER_W = R_TOT // NUM_WORKERS
CHUNK = 128
NCHUNK = R_PER_W // CHUNK

_SC_MESH = plsc.VectorSubcoreMesh(core_axis_name="c", subcore_axis_name="s")


def _sc_pool_body(emb_hbm, idx_hbm, out_hbm, idx_v, acc_v, sem_g):
    wid = lax.axis_index("s") * 2 + lax.axis_index("c")

    zero = jnp.zeros((16,), jnp.float32)

    @pl.loop(0, NCHUNK)
    def _chunk(c):
        pltpu.sync_copy(idx_hbm.at[wid, c], idx_v)

        @pl.loop(0, CHUNK)
        def _zero(i):
            for j in range(EMBED_DIM // 16):
                acc_v[i, pl.ds(j * 16, 16)] = zero

        @pl.loop(0, HIST)
        def _fire(h):
            pltpu.async_copy(emb_hbm.at[idx_v.at[h]], acc_v, sem_g, add=True)

        @pl.loop(0, HIST)
        def _drain(h):
            pltpu.make_async_copy(emb_hbm.at[idx_v.at[h]], acc_v, sem_g).wait()

        row = wid * R_PER_W + c * CHUNK
        pltpu.sync_copy(acc_v, out_hbm.at[pl.ds(row, CHUNK)])


@jax.jit
def _sc_pool(emb, idx4):
    k = pl.kernel(
        _sc_pool_body,
        out_type=jax.ShapeDtypeStruct((R_TOT, EMBED_DIM), jnp.float32),
        mesh=_SC_MESH,
        scratch_types=[
            pltpu.VMEM((HIST, CHUNK), jnp.int32),
            pltpu.VMEM((CHUNK, EMBED_DIM), jnp.float32),
            pltpu.SemaphoreType.DMA,
        ],
        compiler_params=pltpu.CompilerParams(use_tc_tiling_on_sc=False),
    )
    return k(emb, idx4)


BB = 256


def _tc_mlp_body(x_ref, w1_ref, b1_ref, w2_ref, b2_ref, o_ref):
    x = x_ref[...] * (1.0 / BATCH)
    h = jnp.dot(x, w1_ref[...], preferred_element_type=jnp.float32) + b1_ref[...]
    h = jnp.maximum(h, 0.0)
    o = jnp.dot(h, w2_ref[...], preferred_element_type=jnp.float32) + b2_ref[...]
    m = jnp.max(o, axis=1, keepdims=True)
    e = jnp.exp(o - m)
    lse = jnp.log(jnp.sum(e, axis=1, keepdims=True)) + m
    o_ref[...] = o - lse


@jax.jit
def _tc_mlp(x, W1, b1, W2, b2):
    d_in = FEATURE_LEN * EMBED_DIM
    return pl.pallas_call(
        _tc_mlp_body,
        grid=(BATCH // BB,),
        in_specs=[
            pl.BlockSpec((BB, d_in), lambda i: (i, 0)),
            pl.BlockSpec((d_in, LAYER1), lambda i: (0, 0)),
            pl.BlockSpec((1, LAYER1), lambda i: (0, 0)),
            pl.BlockSpec((LAYER1, CLASS_SIZE), lambda i: (0, 0)),
            pl.BlockSpec((1, CLASS_SIZE), lambda i: (0, 0)),
        ],
        out_specs=pl.BlockSpec((BB, CLASS_SIZE), lambda i: (i, 0)),
        out_shape=jax.ShapeDtypeStruct((BATCH, CLASS_SIZE), jnp.float32),
    )(x, W1, b1, W2, b2)


def kernel(batch_inputs, emb, W1, b1, W2, b2):
    idx = batch_inputs.astype(jnp.int32)
    a = idx.transpose(1, 0, 2).reshape(R_TOT, HIST)
    idx4 = a.reshape(NUM_WORKERS, NCHUNK, CHUNK, HIST).swapaxes(2, 3)

    pooled = _sc_pool(emb, idx4)
    x = pooled.reshape(BATCH, FEATURE_LEN * EMBED_DIM)
    return _tc_mlp(x, W1, b1.reshape(1, -1), W2, b2.reshape(1, -1))


# speedup vs baseline: 9.3546x; 1.0964x over previous
import jax
import jax.numpy as jnp
from jax import lax
from jax.experimental import pallas as pl
from jax.experimental.pallas import tpu as pltpu
from jax.experimental.pallas import tpu_sc as plsc

VOCAB = 1000000
EMBED_DIM = 64
FEATURE_LEN = 26
BATCH = 4096
HIST = 50
LAYER1 = 128
CLASS_SIZE = 100

NUM_WORKERS = 32
CHUNK = 128
R_TOT = BATCH * FEATURE_LEN
KTILE = FEATURE_LEN // 2

_SC_MESH = plsc.VectorSubcoreMesh(core_axis_name="c", subcore_axis_name="s")


def _sc_pool_body(emb_hbm, idx_hbm, out_hbm, blk, idxT, acc, dsti,
                  sem_g0, sem_g1):
    wid = lax.axis_index("s") * 2 + lax.axis_index("c")
    b0 = wid * CHUNK
    col16 = lax.iota(jnp.int32, 16)
    zeros16 = jnp.zeros((16,), jnp.float32)

    def prep(par, f):
        pltpu.sync_copy(idx_hbm.at[f, pl.ds(b0, CHUNK)], blk)

        @pl.loop(0, HIST)
        def _t(h):
            hv = jnp.zeros((16,), jnp.int32) + h
            for k in range(CHUNK // 16):
                vals = plsc.load_gather(blk, [col16 + (k * 16), hv])
                idxT[par, h, pl.ds(k * 16, 16)] = vals

    def zero_acc(par):
        @pl.loop(0, CHUNK)
        def _z(i):
            for j in range(EMBED_DIM // 16):
                acc[par, i, pl.ds(j * 16, 16)] = zeros16

    def fire(par, sem):
        @pl.loop(0, HIST)
        def _g(h):
            pltpu.async_copy(emb_hbm.at[idxT.at[par, h]], acc.at[par], sem,
                             add=True)

    def drain(par, sem):
        @pl.loop(0, HIST)
        def _w(h):
            pltpu.make_async_copy(emb_hbm.at[idxT.at[par, h]], acc.at[par],
                                  sem).wait()

    def scatter(par, f):
        base = 2 * b0 + 8192 * (f // 2) + (f % 2)
        for k in range(CHUNK // 16):
            dsti[pl.ds(k * 16, 16)] = (col16 + (k * 16)) * 2 + base
        pltpu.sync_copy(acc.at[par], out_hbm.at[dsti])

    sems = (sem_g0, sem_g1)
    for par in range(2):
        prep(par, par)
        zero_acc(par)
        fire(par, sems[par])

    @pl.loop(0, FEATURE_LEN // 2)
    def _outer(fo):
        for par in range(2):
            f = fo * 2 + par
            drain(par, sems[par])
            scatter(par, f)
            nxt = f + 2

            @pl.when(nxt < FEATURE_LEN)
            def _p():
                prep(par, nxt)
                zero_acc(par)
                fire(par, sems[par])


@jax.jit
def _sc_pool(emb, idx):
    k = pl.kernel(
        _sc_pool_body,
        out_type=jax.ShapeDtypeStruct((R_TOT, EMBED_DIM), jnp.float32),
        mesh=_SC_MESH,
        scratch_types=[
            pltpu.VMEM((CHUNK, HIST), jnp.int32),
            pltpu.VMEM((2, HIST, CHUNK), jnp.int32),
            pltpu.VMEM((2, CHUNK, EMBED_DIM), jnp.float32),
            pltpu.VMEM((CHUNK,), jnp.int32),
            pltpu.SemaphoreType.DMA,
            pltpu.SemaphoreType.DMA,
        ],
        compiler_params=pltpu.CompilerParams(use_tc_tiling_on_sc=False,
                                             needs_layout_passes=False),
    )
    return k(emb, idx)


BB = 256


def _tc_mlp_body(x_ref, w1_ref, b1_ref, w2_ref, b2_ref, o_ref):
    h = jnp.dot(x_ref[0], w1_ref[0], preferred_element_type=jnp.float32)
    for k in range(1, KTILE):
        h += jnp.dot(x_ref[k], w1_ref[k], preferred_element_type=jnp.float32)
    h = h * (1.0 / BATCH) + b1_ref[...]
    h = jnp.maximum(h, 0.0)
    o = jnp.dot(h, w2_ref[...], preferred_element_type=jnp.float32) + b2_ref[...]
    m = jnp.max(o, axis=1, keepdims=True)
    e = jnp.exp(o - m)
    lse = jnp.log(jnp.sum(e, axis=1, keepdims=True)) + m
    o_ref[...] = o - lse


@jax.jit
def _tc_mlp(x3, W13, b1, W2, b2):
    return pl.pallas_call(
        _tc_mlp_body,
        grid=(BATCH // BB,),
        in_specs=[
            pl.BlockSpec((KTILE, BB, LAYER1), lambda i: (0, i, 0)),
            pl.BlockSpec((KTILE, LAYER1, LAYER1), lambda i: (0, 0, 0)),
            pl.BlockSpec((1, LAYER1), lambda i: (0, 0)),
            pl.BlockSpec((LAYER1, CLASS_SIZE), lambda i: (0, 0)),
            pl.BlockSpec((1, CLASS_SIZE), lambda i: (0, 0)),
        ],
        out_specs=pl.BlockSpec((BB, CLASS_SIZE), lambda i: (i, 0)),
        out_shape=jax.ShapeDtypeStruct((BATCH, CLASS_SIZE), jnp.float32),
    )(x3, W13, b1, W2, b2)


def kernel(batch_inputs, emb, W1, b1, W2, b2):
    idx = batch_inputs.astype(jnp.int32)
    pooled = _sc_pool(emb, idx)
    x3 = pooled.reshape(KTILE, BATCH, LAYER1)
    W13 = W1.reshape(KTILE, LAYER1, LAYER1)
    return _tc_mlp(x3, W13, b1.reshape(1, -1), W2, b2.reshape(1, -1))
